# baseline (speedup 1.0000x reference)
import functools

import jax
import jax.numpy as jnp
from jax import lax
from jax.experimental import pallas as pl
from jax.experimental.pallas import tpu as pltpu
from jax.experimental.pallas import tpu_sc as plsc

DIM = 64
NUM_CORES = 1
NUM_WORKERS = NUM_CORES * 16
GATHER_CHUNK = 128
BLOCK = 4096


def _sc_gather_packed(table_t, t):
    batch = t.shape[0]
    n_table = table_t.shape[1]
    b_per_w = batch // NUM_WORKERS
    nchunks = b_per_w // GATHER_CHUNK
    rows_per_tile = 64
    mesh = plsc.VectorSubcoreMesh(
        core_axis_name="c", subcore_axis_name="s", num_cores=NUM_CORES
    )

    @functools.partial(
        pl.kernel,
        mesh=mesh,
        compiler_params=pltpu.CompilerParams(
            use_tc_tiling_on_sc=False,
            skip_device_barrier=True,
            needs_layout_passes=False,
        ),
        out_type=(
            jax.ShapeDtypeStruct((batch // 2, 2 * DIM), jnp.float32),
            jax.ShapeDtypeStruct((n_table, DIM), jnp.float32),
        ),
        scratch_types=[
            pltpu.VMEM((b_per_w,), jnp.int32),
            pltpu.VMEM((b_per_w, DIM), jnp.float32),
            pltpu.VMEM((DIM, rows_per_tile), jnp.float32),
            pltpu.VMEM((rows_per_tile, DIM), jnp.float32),
            pltpu.SemaphoreType.DMA,
            pltpu.SemaphoreType.DMA,
            pltpu.SemaphoreType.DMA,
            pltpu.SemaphoreType.DMA,
        ],
    )
    def gather_kernel(
        tet_hbm, t_hbm, out_hbm, tab_hbm,
        idx_v, rows_v, tcol_v, trow_v, sem_i, sem_g, sem_w, sem_t,
    ):
        wid = lax.axis_index("s") * NUM_CORES + lax.axis_index("c")
        base = wid * b_per_w
        rowbase = 1024 * (base // 2048) + base % 1024
        colbase = DIM * ((base % 2048) // 1024)
        idx_copies = [
            pltpu.async_copy(
                t_hbm.at[pl.ds(base + j * GATHER_CHUNK, GATHER_CHUNK)],
                idx_v.at[pl.ds(j * GATHER_CHUNK, GATHER_CHUNK)],
                sem_i,
            )
            for j in range(nchunks)
        ]

        tab0 = wid * rows_per_tile
        lanes = lax.iota(jnp.int32, 16)

        def transpose_slice(nrows):
            pltpu.async_copy(
                tet_hbm.at[:, pl.ds(tab0, nrows)],
                tcol_v.at[:, pl.ds(0, nrows)],
                sem_t,
            ).wait()

            def body(j, carry):
                jv = jnp.full((16,), 0, jnp.int32) + j
                for g in range(DIM // 16):
                    vals = plsc.load_gather(tcol_v, [lanes + 16 * g, jv])
                    trow_v[j, pl.ds(16 * g, 16)] = vals
                return carry

            lax.fori_loop(0, nrows, body, 0)
            pltpu.async_copy(
                trow_v.at[pl.ds(0, nrows)],
                tab_hbm.at[pl.ds(tab0, nrows)],
                sem_t,
            ).wait()

        full_tiles = n_table // rows_per_tile
        last_rows = n_table - full_tiles * rows_per_tile

        @pl.when(wid < full_tiles)
        def _():
            transpose_slice(rows_per_tile)

        @pl.when(wid == full_tiles)
        def _():
            transpose_slice(last_rows)

        plsc.subcore_barrier()

        gathers = []
        for j in range(nchunks):
            idx_copies[j].wait()
            gathers.append(
                pltpu.async_copy(
                    tab_hbm.at[idx_v.at[pl.ds(j * GATHER_CHUNK, GATHER_CHUNK)]],
                    rows_v.at[pl.ds(j * GATHER_CHUNK, GATHER_CHUNK)],
                    sem_g,
                )
            )
        half = b_per_w // 2
        for j in range(nchunks // 2):
            gathers[j].wait()
        w0 = pltpu.async_copy(
            rows_v.at[pl.ds(0, half)],
            out_hbm.at[pl.ds(rowbase, half), pl.ds(colbase, DIM)],
            sem_w,
        )
        for j in range(nchunks // 2, nchunks):
            gathers[j].wait()
        w1 = pltpu.async_copy(
            rows_v.at[pl.ds(half, half)],
            out_hbm.at[pl.ds(rowbase + half, half), pl.ds(colbase, DIM)],
            sem_w,
        )
        w0.wait()
        w1.wait()

    g128, _ = gather_kernel(table_t, t)
    return g128


def _fused_body(xt_ref, g_ref, w_ref, b_ref, o_ref):
    xw = lax.dot_general(
        w_ref[:, :DIM], xt_ref[...],
        (((1,), (0,)), ((), ())),
        preferred_element_type=jnp.float32,
    )
    bias = b_ref[...]
    for sub in range(BLOCK // 2048):
        yev = lax.dot_general(
            w_ref[:, DIM:], g_ref[sub * 1024:(sub + 1) * 1024, :DIM],
            (((1,), (1,)), ((), ())),
            preferred_element_type=jnp.float32,
        )
        yod = lax.dot_general(
            w_ref[:, DIM:], g_ref[sub * 1024:(sub + 1) * 1024, DIM:],
            (((1,), (1,)), ((), ())),
            preferred_element_type=jnp.float32,
        )
        c0 = sub * 2048
        o_ref[:, c0:c0 + 1024] = xw[:, c0:c0 + 1024] + yev + bias
        o_ref[:, c0 + 1024:c0 + 2048] = xw[:, c0 + 1024:c0 + 2048] + yod + bias


def _fused_matmul_t(xt, g128, W, b_col):
    batch = xt.shape[1]
    grid = batch // BLOCK
    return pl.pallas_call(
        _fused_body,
        grid=(grid,),
        in_specs=[
            pl.BlockSpec((DIM, BLOCK), lambda i: (0, i)),
            pl.BlockSpec((BLOCK // 2, 2 * DIM), lambda i: (i, 0)),
            pl.BlockSpec((DIM, 2 * DIM), lambda i: (0, 0)),
            pl.BlockSpec((DIM, 1), lambda i: (0, 0)),
        ],
        out_specs=pl.BlockSpec((DIM, BLOCK), lambda i: (0, i)),
        out_shape=jax.ShapeDtypeStruct((DIM, batch), jnp.float32),
    )(xt, g128, W, b_col)


def kernel(x, t, time_embed, W, b):
    g128 = _sc_gather_packed(jnp.transpose(time_embed), t.astype(jnp.int32))
    out_t = _fused_matmul_t(
        jnp.transpose(x), g128, W, b.reshape(DIM, 1)
    )
    return jnp.transpose(out_t)


# speedup vs baseline: 1.1284x; 1.1284x over previous
import functools

import jax
import jax.numpy as jnp
from jax import lax
from jax.experimental import pallas as pl
from jax.experimental.pallas import tpu as pltpu
from jax.experimental.pallas import tpu_sc as plsc

DIM = 64
NUM_CORES = 1
NUM_WORKERS = NUM_CORES * 16
GATHER_CHUNK = 128
BLOCK = 4096


def _sc_gather_packed(table, t):
    batch = t.shape[0]
    b_per_w = batch // NUM_WORKERS
    nchunks = b_per_w // GATHER_CHUNK
    mesh = plsc.VectorSubcoreMesh(
        core_axis_name="c", subcore_axis_name="s", num_cores=NUM_CORES
    )

    @functools.partial(
        pl.kernel,
        mesh=mesh,
        compiler_params=pltpu.CompilerParams(
            use_tc_tiling_on_sc=False, skip_device_barrier=True
        ),
        out_type=jax.ShapeDtypeStruct((batch // 2, 2 * DIM), jnp.float32),
        scratch_types=[
            pltpu.VMEM((b_per_w,), jnp.int32),
            pltpu.VMEM((b_per_w, DIM), jnp.float32),
            pltpu.SemaphoreType.DMA,
            pltpu.SemaphoreType.DMA,
            pltpu.SemaphoreType.DMA,
        ],
    )
    def gather_kernel(tab_hbm, t_hbm, out_hbm, idx_v, rows_v, sem_i, sem_g, sem_w):
        wid = lax.axis_index("s") * NUM_CORES + lax.axis_index("c")
        base = wid * b_per_w
        rowbase = 1024 * (base // 2048) + base % 1024
        colbase = DIM * ((base % 2048) // 1024)
        idx_copies = [
            pltpu.async_copy(
                t_hbm.at[pl.ds(base + j * GATHER_CHUNK, GATHER_CHUNK)],
                idx_v.at[pl.ds(j * GATHER_CHUNK, GATHER_CHUNK)],
                sem_i,
            )
            for j in range(nchunks)
        ]
        gathers = []
        for j in range(nchunks):
            idx_copies[j].wait()
            gathers.append(
                pltpu.async_copy(
                    tab_hbm.at[idx_v.at[pl.ds(j * GATHER_CHUNK, GATHER_CHUNK)]],
                    rows_v.at[pl.ds(j * GATHER_CHUNK, GATHER_CHUNK)],
                    sem_g,
                )
            )
        half = b_per_w // 2
        for j in range(nchunks // 2):
            gathers[j].wait()
        w0 = pltpu.async_copy(
            rows_v.at[pl.ds(0, half)],
            out_hbm.at[pl.ds(rowbase, half), pl.ds(colbase, DIM)],
            sem_w,
        )
        for j in range(nchunks // 2, nchunks):
            gathers[j].wait()
        w1 = pltpu.async_copy(
            rows_v.at[pl.ds(half, half)],
            out_hbm.at[pl.ds(rowbase + half, half), pl.ds(colbase, DIM)],
            sem_w,
        )
        w0.wait()
        w1.wait()

    return gather_kernel(table, t)


def _fused_body(xt_ref, g_ref, w_ref, b_ref, o_ref):
    xw = lax.dot_general(
        w_ref[:, :DIM], xt_ref[...],
        (((1,), (0,)), ((), ())),
        preferred_element_type=jnp.float32,
    )
    bias = b_ref[...]
    for sub in range(BLOCK // 2048):
        yev = lax.dot_general(
            w_ref[:, DIM:], g_ref[sub * 1024:(sub + 1) * 1024, :DIM],
            (((1,), (1,)), ((), ())),
            preferred_element_type=jnp.float32,
        )
        yod = lax.dot_general(
            w_ref[:, DIM:], g_ref[sub * 1024:(sub + 1) * 1024, DIM:],
            (((1,), (1,)), ((), ())),
            preferred_element_type=jnp.float32,
        )
        c0 = sub * 2048
        o_ref[:, c0:c0 + 1024] = xw[:, c0:c0 + 1024] + yev + bias
        o_ref[:, c0 + 1024:c0 + 2048] = xw[:, c0 + 1024:c0 + 2048] + yod + bias


def _fused_matmul_t(xt, g128, W, b_col):
    batch = xt.shape[1]
    grid = batch // BLOCK
    return pl.pallas_call(
        _fused_body,
        grid=(grid,),
        in_specs=[
            pl.BlockSpec((DIM, BLOCK), lambda i: (0, i)),
            pl.BlockSpec((BLOCK // 2, 2 * DIM), lambda i: (i, 0)),
            pl.BlockSpec((DIM, 2 * DIM), lambda i: (0, 0)),
            pl.BlockSpec((DIM, 1), lambda i: (0, 0)),
        ],
        out_specs=pl.BlockSpec((DIM, BLOCK), lambda i: (0, i)),
        out_shape=jax.ShapeDtypeStruct((DIM, batch), jnp.float32),
    )(xt, g128, W, b_col)


def kernel(x, t, time_embed, W, b):
    g128 = _sc_gather_packed(time_embed, t.astype(jnp.int32))
    out_t = _fused_matmul_t(
        jnp.transpose(x), g128, W, b.reshape(DIM, 1)
    )
    return jnp.transpose(out_t)


# speedup vs baseline: 1.1632x; 1.0309x over previous
import functools

import jax
import jax.numpy as jnp
from jax import lax
from jax.experimental import pallas as pl
from jax.experimental.pallas import tpu as pltpu
from jax.experimental.pallas import tpu_sc as plsc

DIM = 64
NUM_CORES = 1
NUM_WORKERS = NUM_CORES * 16
GATHER_CHUNK = 128
BLOCK = 8192


def _sc_gather_packed(table, t):
    batch = t.shape[0]
    b_per_w = batch // NUM_WORKERS
    nchunks = b_per_w // GATHER_CHUNK
    mesh = plsc.VectorSubcoreMesh(
        core_axis_name="c", subcore_axis_name="s", num_cores=NUM_CORES
    )

    @functools.partial(
        pl.kernel,
        mesh=mesh,
        compiler_params=pltpu.CompilerParams(
            use_tc_tiling_on_sc=False, skip_device_barrier=True
        ),
        out_type=jax.ShapeDtypeStruct((batch // 2, 2 * DIM), jnp.float32),
        scratch_types=[
            pltpu.VMEM((b_per_w,), jnp.int32),
            pltpu.VMEM((b_per_w, DIM), jnp.float32),
            pltpu.SemaphoreType.DMA,
            pltpu.SemaphoreType.DMA,
            pltpu.SemaphoreType.DMA,
        ],
    )
    def gather_kernel(tab_hbm, t_hbm, out_hbm, idx_v, rows_v, sem_i, sem_g, sem_w):
        wid = lax.axis_index("s") * NUM_CORES + lax.axis_index("c")
        base = wid * b_per_w
        rowbase = 1024 * (base // 2048) + base % 1024
        colbase = DIM * ((base % 2048) // 1024)
        idx_copies = [
            pltpu.async_copy(
                t_hbm.at[pl.ds(base + j * GATHER_CHUNK, GATHER_CHUNK)],
                idx_v.at[pl.ds(j * GATHER_CHUNK, GATHER_CHUNK)],
                sem_i,
            )
            for j in range(nchunks)
        ]
        gathers = []
        for j in range(nchunks):
            idx_copies[j].wait()
            gathers.append(
                pltpu.async_copy(
                    tab_hbm.at[idx_v.at[pl.ds(j * GATHER_CHUNK, GATHER_CHUNK)]],
                    rows_v.at[pl.ds(j * GATHER_CHUNK, GATHER_CHUNK)],
                    sem_g,
                )
            )
        half = b_per_w // 2
        for j in range(nchunks // 2):
            gathers[j].wait()
        w0 = pltpu.async_copy(
            rows_v.at[pl.ds(0, half)],
            out_hbm.at[pl.ds(rowbase, half), pl.ds(colbase, DIM)],
            sem_w,
        )
        for j in range(nchunks // 2, nchunks):
            gathers[j].wait()
        w1 = pltpu.async_copy(
            rows_v.at[pl.ds(half, half)],
            out_hbm.at[pl.ds(rowbase + half, half), pl.ds(colbase, DIM)],
            sem_w,
        )
        w0.wait()
        w1.wait()

    return gather_kernel(table, t)


def _fused_body(xt_ref, g_ref, w_ref, b_ref, o_ref):
    xw = lax.dot_general(
        w_ref[:, :DIM], xt_ref[...],
        (((1,), (0,)), ((), ())),
        preferred_element_type=jnp.float32,
    )
    bias = b_ref[...]
    for sub in range(BLOCK // 2048):
        yev = lax.dot_general(
            w_ref[:, DIM:], g_ref[sub * 1024:(sub + 1) * 1024, :DIM],
            (((1,), (1,)), ((), ())),
            preferred_element_type=jnp.float32,
        )
        yod = lax.dot_general(
            w_ref[:, DIM:], g_ref[sub * 1024:(sub + 1) * 1024, DIM:],
            (((1,), (1,)), ((), ())),
            preferred_element_type=jnp.float32,
        )
        c0 = sub * 2048
        o_ref[:, c0:c0 + 1024] = xw[:, c0:c0 + 1024] + yev + bias
        o_ref[:, c0 + 1024:c0 + 2048] = xw[:, c0 + 1024:c0 + 2048] + yod + bias


def _fused_matmul_t(xt, g128, W, b_col):
    batch = xt.shape[1]
    grid = batch // BLOCK
    return pl.pallas_call(
        _fused_body,
        grid=(grid,),
        in_specs=[
            pl.BlockSpec((DIM, BLOCK), lambda i: (0, i)),
            pl.BlockSpec((BLOCK // 2, 2 * DIM), lambda i: (i, 0)),
            pl.BlockSpec((DIM, 2 * DIM), lambda i: (0, 0)),
            pl.BlockSpec((DIM, 1), lambda i: (0, 0)),
        ],
        out_specs=pl.BlockSpec((DIM, BLOCK), lambda i: (0, i)),
        out_shape=jax.ShapeDtypeStruct((DIM, batch), jnp.float32),
    )(xt, g128, W, b_col)


def kernel(x, t, time_embed, W, b):
    g128 = _sc_gather_packed(time_embed, t.astype(jnp.int32))
    out_t = _fused_matmul_t(
        jnp.transpose(x), g128, W, b.reshape(DIM, 1)
    )
    return jnp.transpose(out_t)
